# baseline (speedup 1.0000x reference)
import jax
import jax.numpy as jnp
from jax import lax
from jax.experimental import pallas as pl
from jax.experimental.pallas import tpu as pltpu

NQ = 8
CB = 1024
CD = 256
RVQ = 512
TT = 512

f32 = jnp.float32
bf16 = jnp.bfloat16


def _mm(a, b):
    return lax.dot_general(a, b, (((1,), (0,)), ((), ())),
                           preferred_element_type=f32)


def _iso(x):
    return jax.lax.optimization_barrier(x)


def _wn(p):
    v, g = p['v'], p['g']
    s = _iso(jnp.sum(v * v, axis=(1, 2), keepdims=True))
    n = _iso(jnp.sqrt(s))
    gv = _iso(g * v)
    return _iso(gv / n)[:, :, 0]


def _split3(x):
    hi = _iso(x.astype(bf16))
    r1 = _iso(x - _iso(hi.astype(f32)))
    mid = _iso(r1.astype(bf16))
    lo = _iso(_iso(r1 - _iso(mid.astype(f32))).astype(bf16))
    return hi, mid, lo


def _main_kernel(zb_ref, wp_ref, bp_ref, a_ref, ba_ref, cbhi2_ref, cb2_ref,
                 cbt0_ref, cbt1_ref, cbt2_ref, wq_ref, bo_ref,
                 wo_ref, bout_ref, out_ref, idx_ref):
    zb = zb_ref[0]
    zp = _mm(wp_ref[...], zb) + bp_ref[...]
    residual = zp
    for k in range(NQ):
        ze = _mm(a_ref[k], residual.astype(bf16)) + ba_ref[k]
        scores2 = _mm(cbhi2_ref[k], ze.astype(bf16))
        zs = ze[0:8]
        acc = zs * zs
        for j in range(1, CD // 8):
            zs = ze[8 * j:8 * (j + 1)]
            acc = acc + zs * zs
        acc = acc[0:4] + acc[4:8]
        acc = acc[0:2] + acc[2:4]
        enc2 = acc[0:1] + acc[1:2]
        dist = (enc2 - scores2) + cb2_ref[k]
        m = jnp.min(dist, axis=0, keepdims=True)
        row = lax.broadcasted_iota(jnp.int32, (CB, TT), 0)
        idx = jnp.min(jnp.where(dist == m, row, CB), axis=0, keepdims=True)
        idx_ref[k, 0] = idx
        oh = (row == idx).astype(bf16)
        g = (_mm(cbt0_ref[k], oh) + _mm(cbt1_ref[k], oh)) \
            + _mm(cbt2_ref[k], oh)
        g = ze + (g - ze)
        dec = _mm(wq_ref[k], g.astype(bf16)) + bo_ref[k]
        residual = residual - dec
    emb = zp - residual
    out_ref[0] = _mm(wo_ref[...], emb.astype(bf16)) + bout_ref[...]


def kernel(z, params):
    B, IN, T = z.shape
    OUT = params['output_proj']['v'].shape[0]
    ip, op = params['input_proj'], params['output_proj']
    qs = params['quantizers']

    wp = _iso(_wn(ip).astype(bf16))
    wo = _iso(_wn(op).astype(bf16))
    a = _iso(jnp.stack([_wn(q['in']) for q in qs]).astype(bf16))
    wq = _iso(jnp.stack([_wn(q['out']) for q in qs]).astype(bf16))
    cb = _iso(jnp.stack([q['codebook'] for q in qs]))
    cb2 = _iso(jnp.sum(cb * cb, axis=2, keepdims=True))
    cb_hi2 = _iso(_iso(cb + cb).astype(bf16))
    t0, t1, t2 = _split3(_iso(jnp.transpose(cb, (0, 2, 1))))

    bp = ip['b'][:, None]
    bout = op['b'][:, None]
    ba = jnp.stack([q['in']['b'] for q in qs])[:, :, None]
    bo = jnp.stack([q['out']['b'] for q in qs])[:, :, None]

    grid = (B, T // TT)
    full = lambda shape: pl.BlockSpec(shape, lambda b, t: (0,) * len(shape))

    out = pl.pallas_call(
        _main_kernel,
        grid=grid,
        in_specs=[
            pl.BlockSpec((1, IN, TT), lambda b, t: (b, 0, t)),
            full((RVQ, IN)), full((RVQ, 1)),
            full((NQ, CD, RVQ)), full((NQ, CD, 1)),
            full((NQ, CB, CD)), full((NQ, CB, 1)),
            full((NQ, CD, CB)), full((NQ, CD, CB)), full((NQ, CD, CB)),
            full((NQ, RVQ, CD)), full((NQ, RVQ, 1)),
            full((OUT, RVQ)), full((OUT, 1)),
        ],
        out_specs=(pl.BlockSpec((1, OUT, TT), lambda b, t: (b, 0, t)),
                   pl.BlockSpec((NQ, 1, 1, TT), lambda b, t: (0, b, 0, t))),
        out_shape=(jax.ShapeDtypeStruct((B, OUT, T), f32),
                   jax.ShapeDtypeStruct((NQ, B, 1, T), jnp.int32)),
    )(z.astype(bf16), wp, bp, a, ba, cb_hi2, cb2, t0, t1, t2, wq, bo,
      wo, bout)
    out, _ = out
    return out


# speedup vs baseline: 1.0233x; 1.0233x over previous
import jax
import jax.numpy as jnp
from jax import lax
from jax.experimental import pallas as pl
from jax.experimental.pallas import tpu as pltpu

NQ = 8
CB = 1024
CD = 256
RVQ = 512
TT = 512

f32 = jnp.float32
bf16 = jnp.bfloat16


def _mm(a, b):
    return lax.dot_general(a, b, (((1,), (0,)), ((), ())),
                           preferred_element_type=f32)


def _iso(x):
    return jax.lax.optimization_barrier(x)


def _wn(p):
    v, g = p['v'], p['g']
    s = _iso(jnp.sum(v * v, axis=(1, 2), keepdims=True))
    n = _iso(jnp.sqrt(s))
    gv = _iso(g * v)
    return _iso(gv / n)[:, :, 0]


def _split3(x):
    hi = _iso(x.astype(bf16))
    r1 = _iso(x - _iso(hi.astype(f32)))
    mid = _iso(r1.astype(bf16))
    lo = _iso(_iso(r1 - _iso(mid.astype(f32))).astype(bf16))
    return hi, mid, lo


def _main_kernel(zb_ref, wp_ref, bp_ref, a_ref, ba_ref, cbhi_ref, cb2_ref,
                 cbt0_ref, cbt1_ref, cbt2_ref, wq_ref, bo_ref,
                 wo_ref, bout_ref, out_ref, idx_ref):
    zb = zb_ref[0]
    zp = _mm(wp_ref[...], zb) + bp_ref[...]
    residual = zp
    for k in range(NQ):
        ze = _mm(a_ref[k], residual.astype(bf16)) + ba_ref[k]
        scores = _mm(cbhi_ref[k], ze.astype(bf16))
        sq = ze * ze
        acc = sq[0:8]
        for j in range(1, CD // 8):
            acc = acc + sq[8 * j:8 * (j + 1)]
        acc = acc[0:4] + acc[4:8]
        acc = acc[0:2] + acc[2:4]
        enc2 = acc[0:1] + acc[1:2]
        dist = (enc2 - 2.0 * scores) + cb2_ref[k]
        m = jnp.min(dist, axis=0, keepdims=True)
        row = lax.broadcasted_iota(jnp.int32, (CB, TT), 0)
        idx = jnp.min(jnp.where(dist == m, row, CB), axis=0, keepdims=True)
        idx_ref[k, 0] = idx
        oh = (row == idx).astype(bf16)
        g = (_mm(cbt0_ref[k], oh) + _mm(cbt1_ref[k], oh)) \
            + _mm(cbt2_ref[k], oh)
        g = ze + (g - ze)
        dec = _mm(wq_ref[k], g.astype(bf16)) + bo_ref[k]
        residual = residual - dec
    emb = zp - residual
    out_ref[0] = _mm(wo_ref[...], emb.astype(bf16)) + bout_ref[...]


def kernel(z, params):
    B, IN, T = z.shape
    OUT = params['output_proj']['v'].shape[0]
    ip, op = params['input_proj'], params['output_proj']
    qs = params['quantizers']

    wp = _iso(_wn(ip).astype(bf16))
    wo = _iso(_wn(op).astype(bf16))
    a = _iso(jnp.stack([_wn(q['in']) for q in qs]).astype(bf16))
    wq = _iso(jnp.stack([_wn(q['out']) for q in qs]).astype(bf16))
    cb = _iso(jnp.stack([q['codebook'] for q in qs]))
    cb2 = _iso(jnp.sum(cb * cb, axis=2, keepdims=True))
    cb_hi = _iso(cb.astype(bf16))
    t0, t1, t2 = _split3(_iso(jnp.transpose(cb, (0, 2, 1))))

    bp = ip['b'][:, None]
    bout = op['b'][:, None]
    ba = jnp.stack([q['in']['b'] for q in qs])[:, :, None]
    bo = jnp.stack([q['out']['b'] for q in qs])[:, :, None]

    grid = (B, T // TT)
    full = lambda shape: pl.BlockSpec(shape, lambda b, t: (0,) * len(shape))

    out = pl.pallas_call(
        _main_kernel,
        grid=grid,
        in_specs=[
            pl.BlockSpec((1, IN, TT), lambda b, t: (b, 0, t)),
            full((RVQ, IN)), full((RVQ, 1)),
            full((NQ, CD, RVQ)), full((NQ, CD, 1)),
            full((NQ, CB, CD)), full((NQ, CB, 1)),
            full((NQ, CD, CB)), full((NQ, CD, CB)), full((NQ, CD, CB)),
            full((NQ, RVQ, CD)), full((NQ, RVQ, 1)),
            full((OUT, RVQ)), full((OUT, 1)),
        ],
        out_specs=(pl.BlockSpec((1, OUT, TT), lambda b, t: (b, 0, t)),
                   pl.BlockSpec((NQ, 1, 1, TT), lambda b, t: (0, b, 0, t))),
        out_shape=(jax.ShapeDtypeStruct((B, OUT, T), f32),
                   jax.ShapeDtypeStruct((NQ, B, 1, T), jnp.int32)),
    )(z.astype(bf16), wp, bp, a, ba, cb_hi, cb2, t0, t1, t2, wq, bo,
      wo, bout)
    out, _ = out
    return out


# speedup vs baseline: 1.2379x; 1.2097x over previous
import jax
import jax.numpy as jnp
from jax import lax
from jax.experimental import pallas as pl
from jax.experimental.pallas import tpu as pltpu

NQ = 8
CB = 1024
CD = 256
RVQ = 512
TT = 1024

f32 = jnp.float32
bf16 = jnp.bfloat16


def _mm(a, b):
    return lax.dot_general(a, b, (((1,), (0,)), ((), ())),
                           preferred_element_type=f32)


def _iso(x):
    return jax.lax.optimization_barrier(x)


def _wn(p):
    v, g = p['v'], p['g']
    s = _iso(jnp.sum(v * v, axis=(1, 2), keepdims=True))
    n = _iso(jnp.sqrt(s))
    gv = _iso(g * v)
    return _iso(gv / n)[:, :, 0]


def _split3(x):
    hi = _iso(x.astype(bf16))
    r1 = _iso(x - _iso(hi.astype(f32)))
    mid = _iso(r1.astype(bf16))
    lo = _iso(_iso(r1 - _iso(mid.astype(f32))).astype(bf16))
    return hi, mid, lo


def _main_kernel(zb_ref, wp_ref, bp_ref, a_ref, ba_ref, cbhi_ref, cb2_ref,
                 cbt0_ref, cbt1_ref, cbt2_ref, wq_ref, bo_ref,
                 wo_ref, bout_ref, out_ref, idx_ref):
    zb = zb_ref[0]
    zp = _mm(wp_ref[...], zb) + bp_ref[...]
    residual = zp
    for k in range(NQ):
        ze = _mm(a_ref[k], residual.astype(bf16)) + ba_ref[k]
        scores = _mm(cbhi_ref[k], ze.astype(bf16))
        sq = ze * ze
        acc = sq[0:8]
        for j in range(1, CD // 8):
            acc = acc + sq[8 * j:8 * (j + 1)]
        acc = acc[0:4] + acc[4:8]
        acc = acc[0:2] + acc[2:4]
        enc2 = acc[0:1] + acc[1:2]
        dist = (enc2 - 2.0 * scores) + cb2_ref[k]
        m = jnp.min(dist, axis=0, keepdims=True)
        row = lax.broadcasted_iota(jnp.int32, (CB, TT), 0)
        idx = jnp.min(jnp.where(dist == m, row, CB), axis=0, keepdims=True)
        idx_ref[k, 0] = idx
        oh = (row == idx).astype(bf16)
        g = (_mm(cbt0_ref[k], oh) + _mm(cbt1_ref[k], oh)) \
            + _mm(cbt2_ref[k], oh)
        g = ze + (g - ze)
        dec = _mm(wq_ref[k], g.astype(bf16)) + bo_ref[k]
        residual = residual - dec
    emb = zp - residual
    out_ref[0] = _mm(wo_ref[...], emb.astype(bf16)) + bout_ref[...]


def kernel(z, params):
    B, IN, T = z.shape
    OUT = params['output_proj']['v'].shape[0]
    ip, op = params['input_proj'], params['output_proj']
    qs = params['quantizers']

    wp = _iso(_wn(ip).astype(bf16))
    wo = _iso(_wn(op).astype(bf16))
    a = _iso(jnp.stack([_wn(q['in']) for q in qs]).astype(bf16))
    wq = _iso(jnp.stack([_wn(q['out']) for q in qs]).astype(bf16))
    cb = _iso(jnp.stack([q['codebook'] for q in qs]))
    cb2 = _iso(jnp.sum(cb * cb, axis=2, keepdims=True))
    cb_hi = _iso(cb.astype(bf16))
    t0, t1, t2 = _split3(_iso(jnp.transpose(cb, (0, 2, 1))))

    bp = ip['b'][:, None]
    bout = op['b'][:, None]
    ba = jnp.stack([q['in']['b'] for q in qs])[:, :, None]
    bo = jnp.stack([q['out']['b'] for q in qs])[:, :, None]

    grid = (B, T // TT)
    full = lambda shape: pl.BlockSpec(shape, lambda b, t: (0,) * len(shape))

    out = pl.pallas_call(
        _main_kernel,
        grid=grid,
        in_specs=[
            pl.BlockSpec((1, IN, TT), lambda b, t: (b, 0, t)),
            full((RVQ, IN)), full((RVQ, 1)),
            full((NQ, CD, RVQ)), full((NQ, CD, 1)),
            full((NQ, CB, CD)), full((NQ, CB, 1)),
            full((NQ, CD, CB)), full((NQ, CD, CB)), full((NQ, CD, CB)),
            full((NQ, RVQ, CD)), full((NQ, RVQ, 1)),
            full((OUT, RVQ)), full((OUT, 1)),
        ],
        out_specs=(pl.BlockSpec((1, OUT, TT), lambda b, t: (b, 0, t)),
                   pl.BlockSpec((NQ, 1, 1, TT), lambda b, t: (0, b, 0, t))),
        out_shape=(jax.ShapeDtypeStruct((B, OUT, T), f32),
                   jax.ShapeDtypeStruct((NQ, B, 1, T), jnp.int32)),
    )(z.astype(bf16), wp, bp, a, ba, cb_hi, cb2, t0, t1, t2, wq, bo,
      wo, bout)
    out, _ = out
    return out
